# baseline (speedup 1.0000x reference)
import functools

import numpy as np
import jax
import jax.numpy as jnp
from jax.experimental import pallas as pl
from jax.experimental.pallas import tpu as pltpu
from jax.experimental.pallas import tpu_sc as plsc

S, D, E, K = 2048, 768, 8, 2
HR, HE = 768, 1536
DC = 2 * D
TS = 256
NT = S // TS
GT = 256
NP = S * K
NGT = NP // GT + E
PM = NGT * GT
OUT2 = K * S + GT
GW = 128
D2 = D // 2


def _pos_encoding():
    pos = np.arange(S)[:, None].astype(np.float32)
    div = np.exp(np.arange(0, D, 2).astype(np.float32) * (-np.log(10000.0) / D))
    pe = np.zeros((S, D), dtype=np.float32)
    pe[:, 0::2] = np.sin(pos * div)
    pe[:, 1::2] = np.cos(pos * div)
    return jnp.asarray(pe)


def _ln(x, g, b):
    m = jnp.mean(x, axis=-1, keepdims=True)
    v = jnp.mean((x - m) ** 2, axis=-1, keepdims=True)
    return (x - m) * jax.lax.rsqrt(v + 1e-5) * g + b


def _bf16_dot(a, b):
    return jnp.dot(a.astype(jnp.bfloat16), b.astype(jnp.bfloat16),
                   preferred_element_type=jnp.float32)


def _router_kernel(x_ref, pe_ref, g_ref, b_ref, w1_ref, b1_ref, w2_ref, b2_ref,
                   t_ref, xp_ref, ti_ref, tw_ref):
    xp = x_ref[...] + pe_ref[...]
    xp_ref[...] = xp
    h = _ln(xp, g_ref[...], b_ref[...])
    a = jax.nn.gelu(_bf16_dot(h, w1_ref[...]) + b1_ref[...])
    logits = _bf16_dot(a, w2_ref[...]) + b2_ref[...]
    l = logits / t_ref[0, 0]
    m = jnp.max(l, axis=-1, keepdims=True)
    p = jnp.exp(l - m)
    probs = p / jnp.sum(p, axis=-1, keepdims=True)
    iota = jax.lax.broadcasted_iota(jnp.int32, (TS, E), 1)
    i1 = jnp.argmax(probs, axis=-1)[:, None]
    m1 = jnp.max(probs, axis=-1, keepdims=True)
    probs2 = jnp.where(iota == i1, -jnp.inf, probs)
    i2 = jnp.argmax(probs2, axis=-1)[:, None]
    m2 = jnp.max(probs2, axis=-1, keepdims=True)
    s = m1 + m2 + 1e-9
    ti_ref[...] = jnp.concatenate([i1, i2], axis=1)
    tw_ref[...] = jnp.concatenate([m1 / s, m2 / s], axis=1)


def _grouped_ffn_kernel(te_ref, gx_ref, wr_ref, w1_ref, b1_ref, w2_ref, b2_ref,
                        y_ref):
    a = jax.nn.gelu(_bf16_dot(gx_ref[...], w1_ref[0]) + b1_ref[0])
    y = _bf16_dot(a, w2_ref[0]) + b2_ref[0]
    y_ref[...] = y * wr_ref[...]


def _combiner_kernel(ca_ref, cb_ref, xp_ref, cg_ref, cbb_ref, w1_ref, b1_ref,
                     w2_ref, b2_ref, og_ref, ob_ref, out_ref):
    comb = ca_ref[...] + cb_ref[...]
    ch = _ln(comb, cg_ref[...], cbb_ref[...])
    a = jax.nn.gelu(_bf16_dot(ch, w1_ref[...]) + b1_ref[...])
    c = _bf16_dot(a, w2_ref[...]) + b2_ref[...]
    out_ref[...] = _ln(xp_ref[...] + c, og_ref[...], ob_ref[...])


_SC_MESH = plsc.VectorSubcoreMesh(core_axis_name="core",
                                  subcore_axis_name="subcore")


def _sc_gather(xpf, rows2):
    @pl.kernel(out_type=jax.ShapeDtypeStruct((2 * PM, D2), jnp.float32),
               mesh=_SC_MESH)
    def k(x_hbm, i_hbm, o_hbm):
        def body(i_vmem, o_vmem):
            pltpu.sync_copy(x_hbm.at[i_vmem.at[0]], o_vmem)

        pltpu.emit_pipeline(
            body,
            grid=(2 * PM // GW,),
            in_specs=[pl.BlockSpec((1, GW), index_map=lambda i: (0, i))],
            out_specs=[pl.BlockSpec((GW, D2), index_map=lambda i: (i, 0))],
            core_axis_name=("core", "subcore"),
            dimension_semantics=(pltpu.PARALLEL,),
        )(i_hbm, o_hbm)

    return k(xpf, rows2)


def _sc_scatter(y2, dest2):
    @pl.kernel(out_type=jax.ShapeDtypeStruct((2 * OUT2, D2), jnp.float32),
               mesh=_SC_MESH)
    def k(y_hbm, i_hbm, o_hbm):
        def body(y_vmem, i_vmem):
            pltpu.sync_copy(y_vmem, o_hbm.at[i_vmem.at[0]])

        pltpu.emit_pipeline(
            body,
            grid=(2 * PM // GW,),
            in_specs=[pl.BlockSpec((GW, D2), index_map=lambda i: (i, 0)),
                      pl.BlockSpec((1, GW), index_map=lambda i: (0, i))],
            out_specs=[],
            core_axis_name=("core", "subcore"),
            dimension_semantics=(pltpu.PARALLEL,),
        )(y_hbm, i_hbm)

    return k(y2, dest2)


def kernel(x, rln_g, rln_b, rW1, rb1, rW2, rb2, temp, eW1, eb1, eW2, eb2,
           cln_g, cln_b, cW1, cb1, cW2, cb2, oln_g, oln_b):
    x2 = x.reshape(S, D)
    pe = _pos_encoding()
    row = lambda v: v.reshape(1, -1)

    xp, topi, topw = pl.pallas_call(
        _router_kernel,
        grid=(NT,),
        in_specs=[
            pl.BlockSpec((TS, D), lambda i: (i, 0)),
            pl.BlockSpec((TS, D), lambda i: (i, 0)),
            pl.BlockSpec((1, D), lambda i: (0, 0)),
            pl.BlockSpec((1, D), lambda i: (0, 0)),
            pl.BlockSpec((D, HR), lambda i: (0, 0)),
            pl.BlockSpec((1, HR), lambda i: (0, 0)),
            pl.BlockSpec((HR, E), lambda i: (0, 0)),
            pl.BlockSpec((1, E), lambda i: (0, 0)),
            pl.BlockSpec((1, 1), lambda i: (0, 0)),
        ],
        out_specs=[
            pl.BlockSpec((TS, D), lambda i: (i, 0)),
            pl.BlockSpec((TS, K), lambda i: (i, 0)),
            pl.BlockSpec((TS, K), lambda i: (i, 0)),
        ],
        out_shape=[
            jax.ShapeDtypeStruct((S, D), jnp.float32),
            jax.ShapeDtypeStruct((S, K), jnp.int32),
            jax.ShapeDtypeStruct((S, K), jnp.float32),
        ],
    )(x2, pe, row(rln_g), row(rln_b), rW1, row(rb1), rW2, row(rb2),
      temp.reshape(1, 1))

    e_p = topi.reshape(NP)
    w_p = topw.reshape(NP)
    t_p = jnp.arange(NP, dtype=jnp.int32) // K
    k_p = jnp.arange(NP, dtype=jnp.int32) % K
    order = jnp.argsort(e_p, stable=True)
    e_s = e_p[order]
    counts = jnp.bincount(e_p, length=E)
    start = jnp.cumsum(counts) - counts
    pc = ((counts + GT - 1) // GT) * GT
    pstart = jnp.cumsum(pc) - pc
    ppos = (pstart[e_s] + jnp.arange(NP, dtype=jnp.int32)
            - start[e_s]).astype(jnp.int32)
    rows = jnp.zeros((PM,), jnp.int32).at[ppos].set(t_p[order])
    wrow = jnp.zeros((PM,), jnp.float32).at[ppos].set(w_p[order])
    dest = jnp.full((PM,), K * S, jnp.int32).at[ppos].set(
        k_p[order] * S + t_p[order])
    tile_expert = jnp.minimum(
        jnp.searchsorted(jnp.cumsum(pc),
                         jnp.arange(NGT, dtype=jnp.int32) * GT, side="right"),
        E - 1).astype(jnp.int32)

    half = jnp.arange(2, dtype=jnp.int32)
    rows2 = (rows[:, None] * 2 + half).reshape(1, 2 * PM)
    gx = _sc_gather(xp.reshape(2 * S, D2), rows2).reshape(PM, D)

    y = pl.pallas_call(
        _grouped_ffn_kernel,
        grid_spec=pltpu.PrefetchScalarGridSpec(
            num_scalar_prefetch=1,
            grid=(NGT,),
            in_specs=[
                pl.BlockSpec((GT, D), lambda i, te: (i, 0)),
                pl.BlockSpec((GT, 1), lambda i, te: (i, 0)),
                pl.BlockSpec((1, D, HE), lambda i, te: (te[i], 0, 0)),
                pl.BlockSpec((1, 1, HE), lambda i, te: (te[i], 0, 0)),
                pl.BlockSpec((1, HE, D), lambda i, te: (te[i], 0, 0)),
                pl.BlockSpec((1, 1, D), lambda i, te: (te[i], 0, 0)),
            ],
            out_specs=pl.BlockSpec((GT, D), lambda i, te: (i, 0)),
        ),
        out_shape=jax.ShapeDtypeStruct((PM, D), jnp.float32),
    )(tile_expert, gx, wrow.reshape(PM, 1), eW1.astype(jnp.bfloat16),
      eb1.reshape(E, 1, HE), eW2.astype(jnp.bfloat16), eb2.reshape(E, 1, D))

    dest2 = (dest[:, None] * 2 + half).reshape(1, 2 * PM)
    out2 = _sc_scatter(y.reshape(2 * PM, D2), dest2).reshape(OUT2, D)

    out = pl.pallas_call(
        _combiner_kernel,
        grid=(NT,),
        in_specs=[
            pl.BlockSpec((TS, D), lambda i: (i, 0)),
            pl.BlockSpec((TS, D), lambda i: (i + NT, 0)),
            pl.BlockSpec((TS, D), lambda i: (i, 0)),
            pl.BlockSpec((1, D), lambda i: (0, 0)),
            pl.BlockSpec((1, D), lambda i: (0, 0)),
            pl.BlockSpec((D, DC), lambda i: (0, 0)),
            pl.BlockSpec((1, DC), lambda i: (0, 0)),
            pl.BlockSpec((DC, D), lambda i: (0, 0)),
            pl.BlockSpec((1, D), lambda i: (0, 0)),
            pl.BlockSpec((1, D), lambda i: (0, 0)),
            pl.BlockSpec((1, D), lambda i: (0, 0)),
        ],
        out_specs=pl.BlockSpec((TS, D), lambda i: (i, 0)),
        out_shape=jax.ShapeDtypeStruct((S, D), jnp.float32),
    )(out2, out2, xp, row(cln_g), row(cln_b), cW1, row(cb1), cW2, row(cb2),
      row(oln_g), row(oln_b))

    return out.reshape(1, S, D)
